# baseline (speedup 1.0000x reference)
import jax
import jax.numpy as jnp
from jax import lax
from jax.experimental import pallas as pl
from jax.experimental.pallas import tpu as pltpu
from jax.experimental.pallas import tpu_sc as plsc

N_ROWS = 320000
D = 128
NSEG = 10000
NUM_CORES = 2
NUM_SUBCORES = 16
NW = NUM_CORES * NUM_SUBCORES
SEG_PER_W = 320
OUT_PAD = NW * SEG_PER_W
CHUNK = 256
NCHUNKS = N_ROWS // CHUNK
GROUPS = CHUNK // 16
NT = D // 16
SEARCH_STEPS = 19
NEG_INF = float("-inf")


def _sc_body(data_hbm, ids_hbm, out_hbm, bs_v, ids_v, rows_v, local_out,
             acc_m, sem_i0, sem_i1, sem_r0, sem_r1):
    c = lax.axis_index("c")
    s = lax.axis_index("s")
    w = c * NUM_SUBCORES + s
    s0 = pl.multiple_of(w * SEG_PER_W, SEG_PER_W)

    neg = jnp.full((16,), NEG_INF, dtype=jnp.float32)

    def init_body(i, _):
        base = pl.multiple_of(i * D, D)
        for t in range(NT):
            local_out[pl.ds(base + t * 16, 16)] = neg
        return 0

    lax.fori_loop(0, SEG_PER_W + 1, init_body, 0)

    def search_body(_, st):
        lo_a, hi_a, lo_b, hi_b = st
        mid_a = (lo_a + hi_a) // 2
        mid_b = (lo_b + hi_b) // 2
        base_a = pl.multiple_of(jnp.minimum(mid_a & ~15, N_ROWS - 16), 16)
        base_b = pl.multiple_of(jnp.minimum(mid_b & ~15, N_ROWS - 16), 16)
        pltpu.sync_copy(ids_hbm.at[pl.ds(base_a, 16)], bs_v.at[0, pl.ds(0, 16)])
        pltpu.sync_copy(ids_hbm.at[pl.ds(base_b, 16)], bs_v.at[1, pl.ds(0, 16)])
        bs_v[0, pl.ds(16, 16)] = bs_v[0, pl.ds(0, 16)]
        bs_v[1, pl.ds(16, 16)] = bs_v[1, pl.ds(0, 16)]
        va = bs_v[0, pl.ds(mid_a - base_a, 16)][0]
        vb = bs_v[1, pl.ds(mid_b - base_b, 16)][0]
        act_a = lo_a < hi_a
        act_b = lo_b < hi_b
        ge_a = va >= s0
        ge_b = vb >= s0 + SEG_PER_W
        lo_a = jnp.where(act_a & ~ge_a, mid_a + 1, lo_a)
        hi_a = jnp.where(act_a & ge_a, mid_a, hi_a)
        lo_b = jnp.where(act_b & ~ge_b, mid_b + 1, lo_b)
        hi_b = jnp.where(act_b & ge_b, mid_b, hi_b)
        return (lo_a, hi_a, lo_b, hi_b)

    z = jnp.int32(0)
    n = jnp.int32(N_ROWS)
    lo, _, hi, _ = lax.fori_loop(0, SEARCH_STEPS, search_body, (z, n, z, n))

    k0 = lo // CHUNK
    k1 = (hi + CHUNK - 1) // CHUNK
    odd = (k1 - k0) & 1
    up = odd & jnp.where(k1 < NCHUNKS, 1, 0)
    k1 = k1 + up
    k0 = k0 - (odd - up)

    sems_i = (sem_i0, sem_i1)
    sems_r = (sem_r0, sem_r1)

    def copies(k, b):
        base = pl.multiple_of(k * CHUNK, CHUNK)
        return (
            pltpu.make_async_copy(ids_hbm.at[pl.ds(base, CHUNK)],
                                  ids_v.at[b], sems_i[b]),
            pltpu.make_async_copy(data_hbm.at[pl.ds(base, CHUNK), :],
                                  rows_v.at[b], sems_r[b]),
        )

    @pl.when(k0 < k1)
    def _():
        for cp in copies(k0, 0):
            cp.start()

    def loads(b, ridx):
        return [rows_v[b, ridx, pl.ds(t * 16, 16)] for t in range(NT)]

    def process(k, b, prev_in):
        for cp in copies(k, b):
            cp.wait()

        def clamp_base(p):
            j = p - s0
            valid = (j >= 0) & (j < SEG_PER_W)
            return jnp.where(valid, j, SEG_PER_W) * D

        def group_body(g, gc):
            prev, fc, lc, idv = gc
            gbase = pl.multiple_of(g * 16, 16)
            idv_next = ids_v[b, pl.ds(jnp.minimum(gbase + 16, CHUNK - 16), 16)]
            nf = idv_next[0]
            nl = idv_next[15]

            @pl.when(fc == lc)
            def _fast():
                @pl.when(fc != prev)
                def _flush():
                    jb = clamp_base(prev)
                    for t in range(NT):
                        local_out[pl.ds(jb + t * 16, 16)] = \
                            acc_m[pl.ds(t * 16, 16)]
                    for t in range(NT):
                        acc_m[pl.ds(t * 16, 16)] = neg

                rowvals = [loads(b, gbase + r) for r in range(16)]
                for t in range(NT):
                    vals = [rowvals[r][t] for r in range(16)]
                    while len(vals) > 1:
                        vals = [jnp.maximum(vals[2 * i], vals[2 * i + 1])
                                for i in range(len(vals) // 2)]
                    acc_m[pl.ds(t * 16, 16)] = \
                        jnp.maximum(acc_m[pl.ds(t * 16, 16)], vals[0])

            @pl.when(fc != lc)
            def _slow():
                accs = [acc_m[pl.ds(t * 16, 16)] for t in range(NT)]
                p = prev
                for r in range(16):
                    row = loads(b, gbase + r)
                    i = idv[r]
                    changed = i != p

                    @pl.when(changed)
                    def _(accs=tuple(accs), p=p):
                        jb = clamp_base(p)
                        for t in range(NT):
                            local_out[pl.ds(jb + t * 16, 16)] = accs[t]

                    for t in range(NT):
                        accs[t] = jnp.maximum(
                            jnp.where(changed, neg, accs[t]), row[t])
                    p = i
                for t in range(NT):
                    acc_m[pl.ds(t * 16, 16)] = accs[t]

            return (lc, nf, nl, idv_next)

        idv0 = ids_v[b, pl.ds(0, 16)]
        prev_out, _, _, _ = lax.fori_loop(
            0, GROUPS, group_body, (prev_in, idv0[0], idv0[15], idv0))

        jb_f = clamp_base(prev_out)
        for t in range(NT):
            local_out[pl.ds(jb_f + t * 16, 16)] = acc_m[pl.ds(t * 16, 16)]
        return prev_out

    for t in range(NT):
        acc_m[pl.ds(t * 16, 16)] = neg

    init = jnp.int32(-1)

    npairs = (k1 - k0) // 2

    def pair_body(kk, carry):
        cur = carry
        for b in (0, 1):
            k = k0 + kk * 2 + b

            @pl.when(k + 1 < k1)
            def _():
                for cp in copies(k + 1, 1 - b):
                    cp.start()

            cur = process(k, b, cur)
        return cur

    lax.fori_loop(0, npairs, pair_body, init)

    obase = pl.multiple_of(s0 * D, SEG_PER_W * D)
    pltpu.sync_copy(local_out.at[pl.ds(0, SEG_PER_W * D)],
                    out_hbm.at[pl.ds(obase, SEG_PER_W * D)])


@jax.jit
def _segment_max_sc(data, segment_ids):
    mesh = plsc.VectorSubcoreMesh(core_axis_name="c", subcore_axis_name="s")
    f = pl.kernel(
        _sc_body,
        mesh=mesh,
        out_type=jax.ShapeDtypeStruct((OUT_PAD * D,), jnp.float32),
        scratch_types=[
            pltpu.VMEM((2, 32), jnp.int32),
            pltpu.VMEM((2, CHUNK), jnp.int32),
            pltpu.VMEM((2, CHUNK, D), jnp.float32),
            pltpu.VMEM(((SEG_PER_W + 1) * D,), jnp.float32),
            pltpu.VMEM((D,), jnp.float32),
            pltpu.SemaphoreType.DMA,
            pltpu.SemaphoreType.DMA,
            pltpu.SemaphoreType.DMA,
            pltpu.SemaphoreType.DMA,
        ],
    )
    return f(data, segment_ids)


def kernel(data, segment_ids):
    out = _segment_max_sc(data, segment_ids)
    return out.reshape(OUT_PAD, D)[:NSEG]


# speedup vs baseline: 1.1978x; 1.1978x over previous
import jax
import jax.numpy as jnp
from jax import lax
from jax.experimental import pallas as pl
from jax.experimental.pallas import tpu as pltpu
from jax.experimental.pallas import tpu_sc as plsc

N_ROWS = 320000
D = 128
NSEG = 10000
NUM_CORES = 2
NUM_SUBCORES = 16
NW = NUM_CORES * NUM_SUBCORES
SEG_PER_W = 320
OUT_PAD = NW * SEG_PER_W
CHUNK = 256
NCHUNKS = N_ROWS // CHUNK
GROUPS = CHUNK // 16
NT = D // 16
SEARCH_STEPS = 19
NEG_INF = float("-inf")


def _sc_body(data_hbm, ids_hbm, out_hbm, bs_v, ids_v, rows_v, local_out,
             sem_i0, sem_i1, sem_r0, sem_r1):
    c = lax.axis_index("c")
    s = lax.axis_index("s")
    w = c * NUM_SUBCORES + s
    s0 = pl.multiple_of(w * SEG_PER_W, SEG_PER_W)

    neg = jnp.full((16,), NEG_INF, dtype=jnp.float32)

    def init_body(i, _):
        for t in range(NT):
            local_out[i, pl.ds(t * 16, 16)] = neg
        return 0

    lax.fori_loop(0, SEG_PER_W + 1, init_body, 0)

    def search_body(_, st):
        lo_a, hi_a, lo_b, hi_b = st
        mid_a = (lo_a + hi_a) // 2
        mid_b = (lo_b + hi_b) // 2
        base_a = pl.multiple_of(jnp.minimum(mid_a & ~15, N_ROWS - 16), 16)
        base_b = pl.multiple_of(jnp.minimum(mid_b & ~15, N_ROWS - 16), 16)
        pltpu.sync_copy(ids_hbm.at[pl.ds(base_a, 16)], bs_v.at[0, pl.ds(0, 16)])
        pltpu.sync_copy(ids_hbm.at[pl.ds(base_b, 16)], bs_v.at[1, pl.ds(0, 16)])
        bs_v[0, pl.ds(16, 16)] = bs_v[0, pl.ds(0, 16)]
        bs_v[1, pl.ds(16, 16)] = bs_v[1, pl.ds(0, 16)]
        va = bs_v[0, pl.ds(mid_a - base_a, 16)][0]
        vb = bs_v[1, pl.ds(mid_b - base_b, 16)][0]
        act_a = lo_a < hi_a
        act_b = lo_b < hi_b
        ge_a = va >= s0
        ge_b = vb >= s0 + SEG_PER_W
        lo_a = jnp.where(act_a & ~ge_a, mid_a + 1, lo_a)
        hi_a = jnp.where(act_a & ge_a, mid_a, hi_a)
        lo_b = jnp.where(act_b & ~ge_b, mid_b + 1, lo_b)
        hi_b = jnp.where(act_b & ge_b, mid_b, hi_b)
        return (lo_a, hi_a, lo_b, hi_b)

    z = jnp.int32(0)
    n = jnp.int32(N_ROWS)
    lo, _, hi, _ = lax.fori_loop(0, SEARCH_STEPS, search_body, (z, n, z, n))

    k0 = lo // CHUNK
    k1 = (hi + CHUNK - 1) // CHUNK
    odd = (k1 - k0) & 1
    up = odd & jnp.where(k1 < NCHUNKS, 1, 0)
    k1 = k1 + up
    k0 = k0 - (odd - up)

    sems_i = (sem_i0, sem_i1)
    sems_r = (sem_r0, sem_r1)

    def copies(k, b):
        base = pl.multiple_of(k * CHUNK, CHUNK)
        return (
            pltpu.make_async_copy(ids_hbm.at[pl.ds(base, CHUNK)],
                                  ids_v.at[b], sems_i[b]),
            pltpu.make_async_copy(data_hbm.at[pl.ds(base, CHUNK), :],
                                  rows_v.at[b], sems_r[b]),
        )

    @pl.when(k0 < k1)
    def _():
        for cp in copies(k0, 0):
            cp.start()

    def loads(b, ridx):
        return [rows_v[b, ridx, pl.ds(t * 16, 16)] for t in range(NT)]

    def process(k, b, carry):
        for cp in copies(k, b):
            cp.wait()

        def group_body(g, gc):
            prev = gc[0]
            accs = list(gc[1:1 + NT])
            nxt = list(gc[1 + NT:1 + 2 * NT])
            nxt2 = list(gc[1 + 2 * NT:1 + 3 * NT])
            idvec = gc[1 + 3 * NT]
            gbase = pl.multiple_of(g * 16, 16)
            idv_next = ids_v[b, pl.ds(jnp.minimum(gbase + 16, CHUNK - 16), 16)]
            for r in range(16):
                row = nxt
                nxt = nxt2
                nidx = gbase + r + 2
                if r >= 14:
                    nidx = jnp.minimum(nidx, CHUNK - 1)
                nxt2 = loads(b, nidx)
                i = idvec[r]
                changed = i != prev
                j = i - s0
                valid = (j >= 0) & (j < SEG_PER_W)
                jj = jnp.where(valid, j, SEG_PER_W)
                for t in range(NT):
                    a = jnp.maximum(jnp.where(changed, neg, accs[t]), row[t])
                    local_out[jj, pl.ds(t * 16, 16)] = a
                    accs[t] = a
                prev = i
            return (prev,) + tuple(accs) + tuple(nxt) + tuple(nxt2) \
                + (idv_next,)

        gc = lax.fori_loop(0, GROUPS, group_body,
                           carry + tuple(loads(b, 0)) + tuple(loads(b, 1))
                           + (ids_v[b, pl.ds(0, 16)],))
        return gc[:1 + NT]

    init = (jnp.int32(-1),) + tuple(neg for _ in range(NT))

    npairs = (k1 - k0) // 2

    def pair_body(kk, carry):
        cur = carry
        for b in (0, 1):
            k = k0 + kk * 2 + b

            @pl.when(k + 1 < k1)
            def _():
                for cp in copies(k + 1, 1 - b):
                    cp.start()

            cur = process(k, b, cur)
        return cur

    lax.fori_loop(0, npairs, pair_body, init)

    obase = pl.multiple_of(s0, SEG_PER_W)
    pltpu.sync_copy(local_out.at[pl.ds(0, SEG_PER_W)],
                    out_hbm.at[pl.ds(obase, SEG_PER_W)])


@jax.jit
def _segment_max_sc(data, segment_ids):
    mesh = plsc.VectorSubcoreMesh(core_axis_name="c", subcore_axis_name="s")
    f = pl.kernel(
        _sc_body,
        mesh=mesh,
        out_type=jax.ShapeDtypeStruct((OUT_PAD, D), jnp.float32),
        scratch_types=[
            pltpu.VMEM((2, 32), jnp.int32),
            pltpu.VMEM((2, CHUNK), jnp.int32),
            pltpu.VMEM((2, CHUNK, D), jnp.float32),
            pltpu.VMEM((SEG_PER_W + 1, D), jnp.float32),
            pltpu.SemaphoreType.DMA,
            pltpu.SemaphoreType.DMA,
            pltpu.SemaphoreType.DMA,
            pltpu.SemaphoreType.DMA,
        ],
    )
    return f(data, segment_ids)


def kernel(data, segment_ids):
    out = _segment_max_sc(data, segment_ids)
    return out[:NSEG]
